# Initial kernel scaffold; baseline (speedup 1.0000x reference)
import jax
import jax.numpy as jnp
from jax.experimental import pallas as pl


def kernel(x, pos, Wq, bq, Wk, bk, Wv, bv, Wpe, bpe, Wpd, bpd, Wa, ba, Wo, bo):
    raise NotImplementedError("write your pallas kernel here")


# speedup vs baseline: 29.1007x; 29.1007x over previous
import jax
import jax.numpy as jnp
from jax.experimental import pallas as pl
from jax.experimental.pallas import tpu as pltpu

_N = 2048
_D = 256
_K = 16
_RB = 256
_NRB = _N // _RB
_HIGH = jax.lax.Precision.HIGHEST


def _proj_body(x_ref, pos_ref, Wq_ref, bq_ref, Wk_ref, bk_ref, Wv_ref, bv_ref,
               Wpe_ref, bpe_ref, Wpd_ref, wa_ref,
               C_ref, V_ref, QW_ref, U_ref):
    x = x_ref[0]
    pos = pos_ref[0]
    pe = (pos[:, 0:1] * Wpe_ref[0:1, :]
          + pos[:, 1:2] * Wpe_ref[1:2, :]
          + pos[:, 2:3] * Wpe_ref[2:3, :]) + bpe_ref[...]
    q = jnp.dot(x, Wq_ref[...], preferred_element_type=jnp.float32,
                precision=_HIGH) + bq_ref[...]
    k = jnp.dot(x, Wk_ref[...], preferred_element_type=jnp.float32,
                precision=_HIGH) + bk_ref[...]
    v = jnp.dot(x, Wv_ref[...], preferred_element_type=jnp.float32,
                precision=_HIGH) + bv_ref[...]
    qw = (q + pe) * wa_ref[...] * jnp.float32(1.0 / 16.0)
    C_ref[0] = k + pe
    V_ref[0] = v
    QW_ref[0] = qw
    u0 = jnp.sum(qw * Wpd_ref[0:1, :], axis=1, keepdims=True)
    u1 = jnp.sum(qw * Wpd_ref[1:2, :], axis=1, keepdims=True)
    u2 = jnp.sum(qw * Wpd_ref[2:3, :], axis=1, keepdims=True)
    U_ref[0] = jnp.concatenate([u0, u1, u2], axis=1)


def _attn_body(pos_ref, post_ref, C_ref, V_ref, QW_ref, U_ref, x_ref,
               Wo_ref, bo_ref, out_ref):
    posb = pos_ref[0]
    post = post_ref[0]
    d0 = posb[:, 0:1] - post[0:1, :]
    d1 = posb[:, 1:2] - post[1:2, :]
    d2 = posb[:, 2:3] - post[2:3, :]
    dist = d0 * d0 + d1 * d1 + d2 * d2

    w = dist
    for _ in range(_K):
        m = jnp.min(w, axis=1, keepdims=True)
        w = jnp.where(w <= m, jnp.float32(jnp.inf), w)
    mask = w == jnp.float32(jnp.inf)

    qwb = QW_ref[0]
    logits = jax.lax.dot_general(
        qwb, C_ref[0], (((1,), (1,)), ((), ())),
        preferred_element_type=jnp.float32, precision=_HIGH)
    ub = U_ref[0]
    logits = (logits
              + ub[:, 0:1] * post[0:1, :]
              + ub[:, 1:2] * post[1:2, :]
              + ub[:, 2:3] * post[2:3, :])
    logits = jnp.where(mask, logits, jnp.float32(-1e30))
    mx = jnp.max(logits, axis=1, keepdims=True)
    p = jnp.exp(logits - mx)
    p = p / jnp.sum(p, axis=1, keepdims=True)
    outp = jnp.dot(p, V_ref[0], preferred_element_type=jnp.float32,
                   precision=_HIGH)
    outp = jnp.dot(outp, Wo_ref[...], preferred_element_type=jnp.float32,
                   precision=_HIGH) + bo_ref[...]
    out_ref[0] = x_ref[0] + jax.nn.gelu(outp)


def kernel(x, pos, Wq, bq, Wk, bk, Wv, bv, Wpe, bpe, Wpd, bpd, Wa, ba, Wo, bo):
    B, S, N, D = x.shape
    x2 = x.reshape(B, N, D)
    pos2 = pos.reshape(B, N, 3)
    pos_t = pos2.transpose(0, 2, 1)
    wa = Wa.reshape(1, D)

    grid = (B, _NRB)
    full = lambda b, r: (b, 0, 0)
    blk = lambda b, r: (b, r, 0)
    wfull = lambda b, r: (0, 0)

    C, V, QW, U = pl.pallas_call(
        _proj_body,
        grid=grid,
        in_specs=[
            pl.BlockSpec((1, _RB, D), blk),
            pl.BlockSpec((1, _RB, 3), blk),
            pl.BlockSpec((D, D), wfull),
            pl.BlockSpec((1, D), wfull),
            pl.BlockSpec((D, D), wfull),
            pl.BlockSpec((1, D), wfull),
            pl.BlockSpec((D, D), wfull),
            pl.BlockSpec((1, D), wfull),
            pl.BlockSpec((3, D), wfull),
            pl.BlockSpec((1, D), wfull),
            pl.BlockSpec((3, D), wfull),
            pl.BlockSpec((1, D), wfull),
        ],
        out_specs=[
            pl.BlockSpec((1, _RB, D), blk),
            pl.BlockSpec((1, _RB, D), blk),
            pl.BlockSpec((1, _RB, D), blk),
            pl.BlockSpec((1, _RB, 3), blk),
        ],
        out_shape=[
            jax.ShapeDtypeStruct((B, N, D), jnp.float32),
            jax.ShapeDtypeStruct((B, N, D), jnp.float32),
            jax.ShapeDtypeStruct((B, N, D), jnp.float32),
            jax.ShapeDtypeStruct((B, N, 3), jnp.float32),
        ],
    )(x2, pos2, Wq, bq.reshape(1, D), Wk, bk.reshape(1, D),
      Wv, bv.reshape(1, D), Wpe, bpe.reshape(1, D), Wpd, wa)

    out = pl.pallas_call(
        _attn_body,
        grid=grid,
        in_specs=[
            pl.BlockSpec((1, _RB, 3), blk),
            pl.BlockSpec((1, 3, N), full),
            pl.BlockSpec((1, N, D), full),
            pl.BlockSpec((1, N, D), full),
            pl.BlockSpec((1, _RB, D), blk),
            pl.BlockSpec((1, _RB, 3), blk),
            pl.BlockSpec((1, _RB, D), blk),
            pl.BlockSpec((D, D), wfull),
            pl.BlockSpec((1, D), wfull),
        ],
        out_specs=pl.BlockSpec((1, _RB, D), blk),
        out_shape=jax.ShapeDtypeStruct((B, N, D), jnp.float32),
    )(pos2, pos_t, C, V, QW, U, x2, Wo, bo.reshape(1, D))

    return out.reshape(B, S, N, D)


# speedup vs baseline: 39.5575x; 1.3593x over previous
import jax
import jax.numpy as jnp
from jax.experimental import pallas as pl
from jax.experimental.pallas import tpu as pltpu

_N = 2048
_D = 256
_K = 16
_RB = 256
_NRB = _N // _RB
def _split_bf16(a):
    hi = a.astype(jnp.bfloat16)
    lo = (a - hi.astype(jnp.float32)).astype(jnp.bfloat16)
    return hi, lo


def _dot3(a, b, dims):
    ah, al = _split_bf16(a)
    bh, bl = _split_bf16(b)
    dot = lambda u, v: jax.lax.dot_general(
        u, v, dims, preferred_element_type=jnp.float32)
    return dot(ah, bh) + (dot(ah, bl) + dot(al, bh))


def _mm(a, b):
    return _dot3(a, b, (((1,), (0,)), ((), ())))


def _proj_body(x_ref, pos_ref, Wq_ref, bq_ref, Wk_ref, bk_ref, Wv_ref, bv_ref,
               Wpe_ref, bpe_ref, Wpd_ref, wa_ref,
               C_ref, V_ref, QW_ref, U_ref):
    x = x_ref[0]
    pos = pos_ref[0]
    pe = (pos[:, 0:1] * Wpe_ref[0:1, :]
          + pos[:, 1:2] * Wpe_ref[1:2, :]
          + pos[:, 2:3] * Wpe_ref[2:3, :]) + bpe_ref[...]
    q = _mm(x, Wq_ref[...]) + bq_ref[...]
    k = _mm(x, Wk_ref[...]) + bk_ref[...]
    v = _mm(x, Wv_ref[...]) + bv_ref[...]
    qw = (q + pe) * wa_ref[...] * jnp.float32(1.0 / 16.0)
    C_ref[0] = k + pe
    V_ref[0] = v
    QW_ref[0] = qw
    u0 = jnp.sum(qw * Wpd_ref[0:1, :], axis=1, keepdims=True)
    u1 = jnp.sum(qw * Wpd_ref[1:2, :], axis=1, keepdims=True)
    u2 = jnp.sum(qw * Wpd_ref[2:3, :], axis=1, keepdims=True)
    U_ref[0] = jnp.concatenate([u0, u1, u2], axis=1)


def _attn_body(pos_ref, post_ref, C_ref, V_ref, QW_ref, U_ref, x_ref,
               Wo_ref, bo_ref, out_ref):
    posb = pos_ref[0]
    post = post_ref[0]
    d0 = posb[:, 0:1] - post[0:1, :]
    d1 = posb[:, 1:2] - post[1:2, :]
    d2 = posb[:, 2:3] - post[2:3, :]
    dist = d0 * d0 + d1 * d1 + d2 * d2

    w = dist
    for _ in range(_K):
        m = jnp.min(w, axis=1, keepdims=True)
        w = jnp.where(w <= m, jnp.float32(jnp.inf), w)
    mask = w == jnp.float32(jnp.inf)

    qwb = QW_ref[0]
    logits = _dot3(qwb, C_ref[0], (((1,), (1,)), ((), ())))
    ub = U_ref[0]
    logits = (logits
              + ub[:, 0:1] * post[0:1, :]
              + ub[:, 1:2] * post[1:2, :]
              + ub[:, 2:3] * post[2:3, :])
    logits = jnp.where(mask, logits, jnp.float32(-1e30))
    mx = jnp.max(logits, axis=1, keepdims=True)
    p = jnp.exp(logits - mx)
    p = p / jnp.sum(p, axis=1, keepdims=True)
    outp = _mm(p, V_ref[0])
    outp = _mm(outp, Wo_ref[...]) + bo_ref[...]
    out_ref[0] = x_ref[0] + jax.nn.gelu(outp)


def kernel(x, pos, Wq, bq, Wk, bk, Wv, bv, Wpe, bpe, Wpd, bpd, Wa, ba, Wo, bo):
    B, S, N, D = x.shape
    x2 = x.reshape(B, N, D)
    pos2 = pos.reshape(B, N, 3)
    pos_t = pos2.transpose(0, 2, 1)
    wa = Wa.reshape(1, D)

    grid = (B, _NRB)
    full = lambda b, r: (b, 0, 0)
    blk = lambda b, r: (b, r, 0)
    wfull = lambda b, r: (0, 0)

    C, V, QW, U = pl.pallas_call(
        _proj_body,
        grid=grid,
        in_specs=[
            pl.BlockSpec((1, _RB, D), blk),
            pl.BlockSpec((1, _RB, 3), blk),
            pl.BlockSpec((D, D), wfull),
            pl.BlockSpec((1, D), wfull),
            pl.BlockSpec((D, D), wfull),
            pl.BlockSpec((1, D), wfull),
            pl.BlockSpec((D, D), wfull),
            pl.BlockSpec((1, D), wfull),
            pl.BlockSpec((3, D), wfull),
            pl.BlockSpec((1, D), wfull),
            pl.BlockSpec((3, D), wfull),
            pl.BlockSpec((1, D), wfull),
        ],
        out_specs=[
            pl.BlockSpec((1, _RB, D), blk),
            pl.BlockSpec((1, _RB, D), blk),
            pl.BlockSpec((1, _RB, D), blk),
            pl.BlockSpec((1, _RB, 3), blk),
        ],
        out_shape=[
            jax.ShapeDtypeStruct((B, N, D), jnp.float32),
            jax.ShapeDtypeStruct((B, N, D), jnp.float32),
            jax.ShapeDtypeStruct((B, N, D), jnp.float32),
            jax.ShapeDtypeStruct((B, N, 3), jnp.float32),
        ],
        compiler_params=pltpu.CompilerParams(
            dimension_semantics=("parallel", "parallel")),
    )(x2, pos2, Wq, bq.reshape(1, D), Wk, bk.reshape(1, D),
      Wv, bv.reshape(1, D), Wpe, bpe.reshape(1, D), Wpd, wa)

    out = pl.pallas_call(
        _attn_body,
        grid=grid,
        in_specs=[
            pl.BlockSpec((1, _RB, 3), blk),
            pl.BlockSpec((1, 3, N), full),
            pl.BlockSpec((1, N, D), full),
            pl.BlockSpec((1, N, D), full),
            pl.BlockSpec((1, _RB, D), blk),
            pl.BlockSpec((1, _RB, 3), blk),
            pl.BlockSpec((1, _RB, D), blk),
            pl.BlockSpec((D, D), wfull),
            pl.BlockSpec((1, D), wfull),
        ],
        out_specs=pl.BlockSpec((1, _RB, D), blk),
        out_shape=jax.ShapeDtypeStruct((B, N, D), jnp.float32),
        compiler_params=pltpu.CompilerParams(
            dimension_semantics=("parallel", "parallel")),
    )(pos2, pos_t, C, V, QW, U, x2, Wo, bo.reshape(1, D))

    return out.reshape(B, S, N, D)
